# Initial kernel scaffold; baseline (speedup 1.0000x reference)
import jax
import jax.numpy as jnp
from jax.experimental import pallas as pl


def kernel(x, edge_index, batch, W1, b1, g1, be1, W2, b2, g2, be2, W3, b3, g3, be3):
    raise NotImplementedError("write your pallas kernel here")


# speedup vs baseline: 23.0040x; 23.0040x over previous
import functools

import jax
import jax.numpy as jnp
from jax import lax
from jax.experimental import pallas as pl
from jax.experimental.pallas import tpu as pltpu
from jax.experimental.pallas import tpu_sc as plsc

N = 10000
E = 320000
D = 128
H = D // 2
G = 64

NC = 2
NS = 16
EPT = E // NS
K = 125
CH = EPT // K
WB = 624
TAIL = N - WB * NS
ZB = 48

_mesh = plsc.VectorSubcoreMesh(core_axis_name="c", subcore_axis_name="s")
_sc_params = pltpu.CompilerParams(use_tc_tiling_on_sc=False)


def _zero_rows(buf, nrows, ncols):
    def body(i, _):
        for j in range(ncols // 16):
            buf[i, pl.ds(j * 16, 16)] = jnp.zeros((16,), jnp.float32)
        return 0
    lax.fori_loop(0, nrows, body, 0)


@functools.partial(
    pl.kernel,
    mesh=_mesh,
    out_type=jax.ShapeDtypeStruct((NC, N, H), jnp.float32),
    scratch_types=[
        pltpu.VMEM((CH, K), jnp.int32),
        pltpu.VMEM((CH, K), jnp.int32),
        pltpu.VMEM((K, H), jnp.float32),
        pltpu.VMEM((K, H), jnp.float32),
        pltpu.VMEM_SHARED((N, H), jnp.float32),
        pltpu.SemaphoreType.DMA,
        pltpu.SemaphoreType.DMA,
    ],
    compiler_params=_sc_params,
)
def _sc_propagate(hsA_hbm, hsB_hbm, src_hbm, dst_hbm, out_hbm,
                  src_v, dst_v, rows0, rows1, acc, sem0, sem1):
    c = lax.axis_index("c")
    s = lax.axis_index("s")

    pltpu.sync_copy(src_hbm.at[s], src_v)
    pltpu.sync_copy(dst_hbm.at[s], dst_v)

    _zero_rows(rows0, ZB, H)
    for k in range(WB // ZB):
        pltpu.sync_copy(rows0.at[pl.ds(0, ZB)],
                        acc.at[pl.ds(s * WB + k * ZB, ZB)])

    @pl.when(s == 0)
    def _():
        pltpu.sync_copy(rows0.at[pl.ds(0, TAIL)], acc.at[pl.ds(WB * NS, TAIL)])
    plsc.subcore_barrier()

    def run(hs_hbm):
        pltpu.async_copy(hs_hbm.at[src_v.at[0]], rows0, sem0)

        def body(i, _):
            c0 = 2 * i
            pltpu.async_copy(hs_hbm.at[src_v.at[c0 + 1]], rows1, sem1)
            pltpu.make_async_copy(hs_hbm.at[src_v.at[c0]], rows0, sem0).wait()
            pltpu.sync_copy(rows0, acc.at[dst_v.at[c0]], add=True)
            c2 = jnp.minimum(c0 + 2, CH - 1)
            pltpu.async_copy(hs_hbm.at[src_v.at[c2]], rows0, sem0)
            pltpu.make_async_copy(hs_hbm.at[src_v.at[c0 + 1]], rows1,
                                  sem1).wait()
            pltpu.sync_copy(rows1, acc.at[dst_v.at[c0 + 1]], add=True)
            return 0

        lax.fori_loop(0, CH // 2, body, 0)
        pltpu.make_async_copy(hs_hbm.at[src_v.at[0]], rows0, sem0).wait()

    @pl.when(c == 0)
    def _():
        run(hsA_hbm)

    @pl.when(c == 1)
    def _():
        run(hsB_hbm)

    plsc.subcore_barrier()
    pltpu.sync_copy(acc.at[pl.ds(s * WB, WB)],
                    out_hbm.at[c, pl.ds(s * WB, WB), :])

    @pl.when(s == 0)
    def _():
        pltpu.sync_copy(acc.at[pl.ds(WB * NS, TAIL)],
                        out_hbm.at[c, pl.ds(WB * NS, TAIL), :])


@functools.partial(
    pl.kernel,
    mesh=_mesh,
    out_type=jax.ShapeDtypeStruct((NC, N, 16), jnp.float32),
    scratch_types=[
        pltpu.VMEM((CH, K), jnp.int32),
        pltpu.VMEM((K, 16), jnp.float32),
        pltpu.VMEM((K, 16), jnp.float32),
        pltpu.VMEM_SHARED((N, 16), jnp.float32),
    ],
    compiler_params=_sc_params,
)
def _sc_degree(dst_hbm, out_hbm, dst_v, ones_v, zero_v, acc):
    c = lax.axis_index("c")
    s = lax.axis_index("s")

    pltpu.sync_copy(dst_hbm.at[s], dst_v)

    def fill(i, _):
        ones_v[i, pl.ds(0, 16)] = jnp.ones((16,), jnp.float32)
        zero_v[i, pl.ds(0, 16)] = jnp.zeros((16,), jnp.float32)
        return 0
    lax.fori_loop(0, K, fill, 0)
    for k in range(WB // ZB):
        pltpu.sync_copy(zero_v.at[pl.ds(0, ZB)],
                        acc.at[pl.ds(s * WB + k * ZB, ZB)])

    @pl.when(s == 0)
    def _():
        pltpu.sync_copy(zero_v.at[pl.ds(0, TAIL)], acc.at[pl.ds(WB * NS, TAIL)])
    plsc.subcore_barrier()

    def body(i, _):
        pltpu.sync_copy(ones_v, acc.at[dst_v.at[c * (CH // 2) + i]], add=True)
        return 0
    lax.fori_loop(0, CH // 2, body, 0)

    plsc.subcore_barrier()
    pltpu.sync_copy(acc.at[pl.ds(s * WB, WB)],
                    out_hbm.at[c, pl.ds(s * WB, WB), :])

    @pl.when(s == 0)
    def _():
        pltpu.sync_copy(acc.at[pl.ds(WB * NS, TAIL)],
                        out_hbm.at[c, pl.ds(WB * NS, TAIL), :])


def _tc_pre(x_ref, w_ref, degp_ref, hp_ref, hsa_ref, hsb_ref, dinv_ref):
    deg = 1.0 + degp_ref[0, :, 0:1] + degp_ref[1, :, 0:1]
    dinv = lax.rsqrt(deg)
    hp = jnp.dot(x_ref[...], w_ref[...], preferred_element_type=jnp.float32)
    hp_ref[...] = hp
    hs = hp * dinv
    hsa_ref[...] = hs[:, :H]
    hsb_ref[...] = hs[:, H:]
    dinv_ref[...] = dinv


def _bn_input(accp_ref, hp_ref, dinv_ref, b_ref):
    dinv = dinv_ref[...]
    acc = jnp.concatenate([accp_ref[0], accp_ref[1]], axis=1)
    y = acc * dinv + hp_ref[...] * (dinv * dinv) + b_ref[...]
    mu = jnp.mean(y, axis=0, keepdims=True)
    var = jnp.mean((y - mu) ** 2, axis=0, keepdims=True)
    return y, mu, var


def _tc_mid(accp_ref, hp_ref, dinv_ref, b_ref, g_ref, be_ref, w_ref,
            hp2_ref, hsa_ref, hsb_ref):
    y, mu, var = _bn_input(accp_ref, hp_ref, dinv_ref, b_ref)
    h = (y - mu) * lax.rsqrt(var + 1e-5) * g_ref[...] + be_ref[...]
    h = jnp.maximum(h, 0.0)
    hp2 = jnp.dot(h, w_ref[...], preferred_element_type=jnp.float32)
    hp2_ref[...] = hp2
    hs2 = hp2 * dinv_ref[...]
    hsa_ref[...] = hs2[:, :H]
    hsb_ref[...] = hs2[:, H:]


def _tc_post(accp_ref, hp_ref, dinv_ref, b_ref, g_ref, be_ref, batch_ref,
             out_ref):
    y, mu, var = _bn_input(accp_ref, hp_ref, dinv_ref, b_ref)
    h = (y - mu) * lax.rsqrt(var + 1e-5) * g_ref[...] + be_ref[...]
    onehot = (lax.broadcasted_iota(jnp.int32, (G, N), 0)
              == batch_ref[...]).astype(jnp.float32)
    cnt = jnp.sum(onehot, axis=1, keepdims=True)
    pooled = jnp.dot(onehot, h, preferred_element_type=jnp.float32)
    out_ref[...] = pooled / jnp.maximum(cnt, 1.0)


_f32 = jnp.float32


def kernel(x, edge_index, batch, W1, b1, g1, be1, W2, b2, g2, be2,
           W3, b3, g3, be3):
    src3 = edge_index[0].reshape(NS, CH, K).astype(jnp.int32)
    dst3 = edge_index[1].reshape(NS, CH, K).astype(jnp.int32)
    batch2 = batch.reshape(1, N).astype(jnp.int32)
    b1r, b2r, b3r = (v.reshape(1, D) for v in (b1, b2, b3))
    g1r, g2r, g3r = (v.reshape(1, D) for v in (g1, g2, g3))
    be1r, be2r, be3r = (v.reshape(1, D) for v in (be1, be2, be3))

    degp = _sc_degree(dst3)

    hp1, hsa1, hsb1, dinv = pl.pallas_call(
        _tc_pre,
        out_shape=[jax.ShapeDtypeStruct((N, D), _f32),
                   jax.ShapeDtypeStruct((N, H), _f32),
                   jax.ShapeDtypeStruct((N, H), _f32),
                   jax.ShapeDtypeStruct((N, 1), _f32)],
    )(x, W1, degp)

    acc1 = _sc_propagate(hsa1, hsb1, src3, dst3)
    hp2, hsa2, hsb2 = pl.pallas_call(
        _tc_mid,
        out_shape=[jax.ShapeDtypeStruct((N, D), _f32),
                   jax.ShapeDtypeStruct((N, H), _f32),
                   jax.ShapeDtypeStruct((N, H), _f32)],
    )(acc1, hp1, dinv, b1r, g1r, be1r, W2)

    acc2 = _sc_propagate(hsa2, hsb2, src3, dst3)
    hp3, hsa3, hsb3 = pl.pallas_call(
        _tc_mid,
        out_shape=[jax.ShapeDtypeStruct((N, D), _f32),
                   jax.ShapeDtypeStruct((N, H), _f32),
                   jax.ShapeDtypeStruct((N, H), _f32)],
    )(acc2, hp2, dinv, b2r, g2r, be2r, W3)

    acc3 = _sc_propagate(hsa3, hsb3, src3, dst3)
    out = pl.pallas_call(
        _tc_post,
        out_shape=jax.ShapeDtypeStruct((G, D), _f32),
    )(acc3, hp3, dinv, b3r, g3r, be3r, batch2)
    return out


# speedup vs baseline: 27.8917x; 1.2125x over previous
import functools

import jax
import jax.numpy as jnp
from jax import lax
from jax.experimental import pallas as pl
from jax.experimental.pallas import tpu as pltpu
from jax.experimental.pallas import tpu_sc as plsc

N = 10000
E = 320000
D = 128
G = 64

NC = 2
NS = 16
EPT = E // (NC * NS)
K = 100
CH = EPT // K
KD = 125
EPTD = E // NS
CHD = EPTD // KD
WB = 624
TAIL = N - WB * NS
ZB = 48

_mesh = plsc.VectorSubcoreMesh(core_axis_name="c", subcore_axis_name="s")
_sc_params = pltpu.CompilerParams(use_tc_tiling_on_sc=False)


def _zero_rows(buf, nrows, ncols):
    def body(i, _):
        for j in range(ncols // 16):
            buf[i, pl.ds(j * 16, 16)] = jnp.zeros((16,), jnp.float32)
        return 0
    lax.fori_loop(0, nrows, body, 0)


@functools.partial(
    pl.kernel,
    mesh=_mesh,
    out_type=jax.ShapeDtypeStruct((NC, N, D), jnp.float32),
    scratch_types=[
        pltpu.VMEM((CH, K), jnp.int32),
        pltpu.VMEM((CH, K), jnp.int32),
        pltpu.VMEM((2, K, D), jnp.float32),
        pltpu.VMEM_SHARED((N, D), jnp.float32),
        pltpu.SemaphoreType.DMA((2,)),
    ],
    compiler_params=_sc_params,
)
def _sc_propagate(hs_hbm, src_hbm, dst_hbm, out_hbm,
                  src_v, dst_v, rows, acc, sem):
    c = lax.axis_index("c")
    s = lax.axis_index("s")

    pltpu.sync_copy(src_hbm.at[c, s], src_v)
    pltpu.sync_copy(dst_hbm.at[c, s], dst_v)

    _zero_rows(rows.at[0], ZB, D)
    for k in range(WB // ZB):
        pltpu.sync_copy(rows.at[0].at[pl.ds(0, ZB)],
                        acc.at[pl.ds(s * WB + k * ZB, ZB)])

    @pl.when(s == 0)
    def _():
        pltpu.sync_copy(rows.at[0].at[pl.ds(0, TAIL)],
                        acc.at[pl.ds(WB * NS, TAIL)])
    plsc.subcore_barrier()

    pltpu.async_copy(hs_hbm.at[src_v.at[0]], rows.at[0], sem.at[0])

    def body(i, _):
        nxt = jnp.minimum(i + 1, CH - 1)
        pn = lax.rem(i + 1, 2)
        pc = lax.rem(i, 2)
        pltpu.async_copy(hs_hbm.at[src_v.at[nxt]], rows.at[pn], sem.at[pn])
        pltpu.make_async_copy(hs_hbm.at[src_v.at[i]], rows.at[pc],
                              sem.at[pc]).wait()
        pltpu.sync_copy(rows.at[pc], acc.at[dst_v.at[i]], add=True)
        return 0

    lax.fori_loop(0, CH, body, 0)
    pltpu.make_async_copy(hs_hbm.at[src_v.at[CH - 1]], rows.at[CH % 2],
                          sem.at[CH % 2]).wait()

    plsc.subcore_barrier()
    pltpu.sync_copy(acc.at[pl.ds(s * WB, WB)],
                    out_hbm.at[c, pl.ds(s * WB, WB), :])

    @pl.when(s == 0)
    def _():
        pltpu.sync_copy(acc.at[pl.ds(WB * NS, TAIL)],
                        out_hbm.at[c, pl.ds(WB * NS, TAIL), :])


@functools.partial(
    pl.kernel,
    mesh=_mesh,
    out_type=jax.ShapeDtypeStruct((NC, N, 16), jnp.float32),
    scratch_types=[
        pltpu.VMEM((CHD, KD), jnp.int32),
        pltpu.VMEM((KD, 16), jnp.float32),
        pltpu.VMEM((KD, 16), jnp.float32),
        pltpu.VMEM_SHARED((N, 16), jnp.float32),
    ],
    compiler_params=_sc_params,
)
def _sc_degree(dst_hbm, out_hbm, dst_v, ones_v, zero_v, acc):
    c = lax.axis_index("c")
    s = lax.axis_index("s")

    pltpu.sync_copy(dst_hbm.at[s], dst_v)

    def fill(i, _):
        ones_v[i, pl.ds(0, 16)] = jnp.ones((16,), jnp.float32)
        zero_v[i, pl.ds(0, 16)] = jnp.zeros((16,), jnp.float32)
        return 0
    lax.fori_loop(0, KD, fill, 0)
    for k in range(WB // ZB):
        pltpu.sync_copy(zero_v.at[pl.ds(0, ZB)],
                        acc.at[pl.ds(s * WB + k * ZB, ZB)])

    @pl.when(s == 0)
    def _():
        pltpu.sync_copy(zero_v.at[pl.ds(0, TAIL)], acc.at[pl.ds(WB * NS, TAIL)])
    plsc.subcore_barrier()

    def body(i, _):
        pltpu.sync_copy(ones_v, acc.at[dst_v.at[c * (CHD // 2) + i]], add=True)
        return 0
    lax.fori_loop(0, CHD // 2, body, 0)

    plsc.subcore_barrier()
    pltpu.sync_copy(acc.at[pl.ds(s * WB, WB)],
                    out_hbm.at[c, pl.ds(s * WB, WB), :])

    @pl.when(s == 0)
    def _():
        pltpu.sync_copy(acc.at[pl.ds(WB * NS, TAIL)],
                        out_hbm.at[c, pl.ds(WB * NS, TAIL), :])


def _tc_pre(x_ref, w_ref, degp_ref, hp_ref, hs_ref, dinv_ref):
    deg = 1.0 + degp_ref[0, :, 0:1] + degp_ref[1, :, 0:1]
    dinv = lax.rsqrt(deg)
    hp = jnp.dot(x_ref[...], w_ref[...], preferred_element_type=jnp.float32)
    hp_ref[...] = hp
    hs_ref[...] = hp * dinv
    dinv_ref[...] = dinv


def _bn_input(accp_ref, hp_ref, dinv_ref, b_ref):
    dinv = dinv_ref[...]
    acc = accp_ref[0] + accp_ref[1]
    y = acc * dinv + hp_ref[...] * (dinv * dinv) + b_ref[...]
    mu = jnp.mean(y, axis=0, keepdims=True)
    var = jnp.mean((y - mu) ** 2, axis=0, keepdims=True)
    return y, mu, var


def _tc_mid(accp_ref, hp_ref, dinv_ref, b_ref, g_ref, be_ref, w_ref,
            hp2_ref, hs2_ref):
    y, mu, var = _bn_input(accp_ref, hp_ref, dinv_ref, b_ref)
    h = (y - mu) * lax.rsqrt(var + 1e-5) * g_ref[...] + be_ref[...]
    h = jnp.maximum(h, 0.0)
    hp2 = jnp.dot(h, w_ref[...], preferred_element_type=jnp.float32)
    hp2_ref[...] = hp2
    hs2_ref[...] = hp2 * dinv_ref[...]


def _tc_post(accp_ref, hp_ref, dinv_ref, b_ref, g_ref, be_ref, batch_ref,
             out_ref):
    y, mu, var = _bn_input(accp_ref, hp_ref, dinv_ref, b_ref)
    h = (y - mu) * lax.rsqrt(var + 1e-5) * g_ref[...] + be_ref[...]
    onehot = (lax.broadcasted_iota(jnp.int32, (G, N), 0)
              == batch_ref[...]).astype(jnp.float32)
    cnt = jnp.sum(onehot, axis=1, keepdims=True)
    pooled = jnp.dot(onehot, h, preferred_element_type=jnp.float32)
    out_ref[...] = pooled / jnp.maximum(cnt, 1.0)


_f32 = jnp.float32


def kernel(x, edge_index, batch, W1, b1, g1, be1, W2, b2, g2, be2,
           W3, b3, g3, be3):
    src4 = edge_index[0].reshape(NC, NS, CH, K).astype(jnp.int32)
    dst4 = edge_index[1].reshape(NC, NS, CH, K).astype(jnp.int32)
    dst3 = edge_index[1].reshape(NS, CHD, KD).astype(jnp.int32)
    batch2 = batch.reshape(1, N).astype(jnp.int32)
    b1r, b2r, b3r = (v.reshape(1, D) for v in (b1, b2, b3))
    g1r, g2r, g3r = (v.reshape(1, D) for v in (g1, g2, g3))
    be1r, be2r, be3r = (v.reshape(1, D) for v in (be1, be2, be3))

    degp = _sc_degree(dst3)

    hp1, hs1, dinv = pl.pallas_call(
        _tc_pre,
        out_shape=[jax.ShapeDtypeStruct((N, D), _f32),
                   jax.ShapeDtypeStruct((N, D), _f32),
                   jax.ShapeDtypeStruct((N, 1), _f32)],
    )(x, W1, degp)

    acc1 = _sc_propagate(hs1, src4, dst4)
    hp2, hs2 = pl.pallas_call(
        _tc_mid,
        out_shape=[jax.ShapeDtypeStruct((N, D), _f32),
                   jax.ShapeDtypeStruct((N, D), _f32)],
    )(acc1, hp1, dinv, b1r, g1r, be1r, W2)

    acc2 = _sc_propagate(hs2, src4, dst4)
    hp3, hs3 = pl.pallas_call(
        _tc_mid,
        out_shape=[jax.ShapeDtypeStruct((N, D), _f32),
                   jax.ShapeDtypeStruct((N, D), _f32)],
    )(acc2, hp2, dinv, b2r, g2r, be2r, W3)

    acc3 = _sc_propagate(hs3, src4, dst4)
    out = pl.pallas_call(
        _tc_post,
        out_shape=jax.ShapeDtypeStruct((G, D), _f32),
    )(acc3, hp3, dinv, b3r, g3r, be3r, batch2)
    return out


# speedup vs baseline: 28.9589x; 1.0383x over previous
import functools

import jax
import jax.numpy as jnp
from jax import lax
from jax.experimental import pallas as pl
from jax.experimental.pallas import tpu as pltpu
from jax.experimental.pallas import tpu_sc as plsc

N = 10000
E = 320000
D = 128
G = 64

NC = 2
NS = 16
EPT = E // (NC * NS)
K = 100
CH = EPT // K
KD = 125
EPTD = E // NS
CHD = EPTD // KD
WB = 624
TAIL = N - WB * NS
ZB = 48

_mesh = plsc.VectorSubcoreMesh(core_axis_name="c", subcore_axis_name="s")
_sc_params = pltpu.CompilerParams(use_tc_tiling_on_sc=False)


def _zero_rows(buf, nrows, ncols):
    def body(i, _):
        for j in range(ncols // 16):
            buf[i, pl.ds(j * 16, 16)] = jnp.zeros((16,), jnp.float32)
        return 0
    lax.fori_loop(0, nrows, body, 0)


@functools.partial(
    pl.kernel,
    mesh=_mesh,
    out_type=jax.ShapeDtypeStruct((NC, N, D), jnp.float32),
    scratch_types=[
        pltpu.VMEM((CH, K), jnp.int32),
        pltpu.VMEM((CH, K), jnp.int32),
        pltpu.VMEM((2, K, D), jnp.float32),
        pltpu.VMEM_SHARED((N, D), jnp.float32),
        pltpu.SemaphoreType.DMA((2,)),
    ],
    compiler_params=_sc_params,
)
def _sc_propagate(hs_hbm, src_hbm, dst_hbm, out_hbm,
                  src_v, dst_v, rows, acc, sem):
    c = lax.axis_index("c")
    s = lax.axis_index("s")

    pltpu.async_copy(src_hbm.at[c, s], src_v, sem.at[0])
    pltpu.async_copy(dst_hbm.at[c, s], dst_v, sem.at[0])
    _zero_rows(rows.at[0], ZB, D)
    for k in range(WB // ZB):
        pltpu.async_copy(rows.at[0].at[pl.ds(0, ZB)],
                         acc.at[pl.ds(s * WB + k * ZB, ZB)], sem.at[1])

    @pl.when(s == 0)
    def _():
        pltpu.async_copy(rows.at[0].at[pl.ds(0, TAIL)],
                         acc.at[pl.ds(WB * NS, TAIL)], sem.at[1])

    pltpu.make_async_copy(src_hbm.at[c, s], src_v, sem.at[0]).wait()
    pltpu.make_async_copy(dst_hbm.at[c, s], dst_v, sem.at[0]).wait()
    for k in range(WB // ZB):
        pltpu.make_async_copy(rows.at[0].at[pl.ds(0, ZB)],
                              acc.at[pl.ds(s * WB + k * ZB, ZB)],
                              sem.at[1]).wait()

    @pl.when(s == 0)
    def _():
        pltpu.make_async_copy(rows.at[0].at[pl.ds(0, TAIL)],
                              acc.at[pl.ds(WB * NS, TAIL)], sem.at[1]).wait()
    plsc.subcore_barrier()

    pltpu.async_copy(hs_hbm.at[src_v.at[0]], rows.at[0], sem.at[0])

    def body(i, _):
        nxt = jnp.minimum(i + 1, CH - 1)
        pn = lax.rem(i + 1, 2)
        pc = lax.rem(i, 2)
        pltpu.async_copy(hs_hbm.at[src_v.at[nxt]], rows.at[pn], sem.at[pn])
        pltpu.make_async_copy(hs_hbm.at[src_v.at[i]], rows.at[pc],
                              sem.at[pc]).wait()
        pltpu.sync_copy(rows.at[pc], acc.at[dst_v.at[i]], add=True)
        return 0

    lax.fori_loop(0, CH, body, 0)
    pltpu.make_async_copy(hs_hbm.at[src_v.at[CH - 1]], rows.at[CH % 2],
                          sem.at[CH % 2]).wait()

    plsc.subcore_barrier()
    pltpu.sync_copy(acc.at[pl.ds(s * WB, WB)],
                    out_hbm.at[c, pl.ds(s * WB, WB), :])

    @pl.when(s == 0)
    def _():
        pltpu.sync_copy(acc.at[pl.ds(WB * NS, TAIL)],
                        out_hbm.at[c, pl.ds(WB * NS, TAIL), :])


@functools.partial(
    pl.kernel,
    mesh=_mesh,
    out_type=jax.ShapeDtypeStruct((NC, N, 16), jnp.float32),
    scratch_types=[
        pltpu.VMEM((CHD, KD), jnp.int32),
        pltpu.VMEM((KD, 16), jnp.float32),
        pltpu.VMEM((KD, 16), jnp.float32),
        pltpu.VMEM_SHARED((N, 16), jnp.float32),
    ],
    compiler_params=_sc_params,
)
def _sc_degree(dst_hbm, out_hbm, dst_v, ones_v, zero_v, acc):
    c = lax.axis_index("c")
    s = lax.axis_index("s")

    pltpu.sync_copy(dst_hbm.at[s], dst_v)

    def fill(i, _):
        ones_v[i, pl.ds(0, 16)] = jnp.ones((16,), jnp.float32)
        zero_v[i, pl.ds(0, 16)] = jnp.zeros((16,), jnp.float32)
        return 0
    lax.fori_loop(0, KD, fill, 0)
    for k in range(WB // ZB):
        pltpu.sync_copy(zero_v.at[pl.ds(0, ZB)],
                        acc.at[pl.ds(s * WB + k * ZB, ZB)])

    @pl.when(s == 0)
    def _():
        pltpu.sync_copy(zero_v.at[pl.ds(0, TAIL)], acc.at[pl.ds(WB * NS, TAIL)])
    plsc.subcore_barrier()

    def body(i, _):
        pltpu.sync_copy(ones_v, acc.at[dst_v.at[c * (CHD // 2) + i]], add=True)
        return 0
    lax.fori_loop(0, CHD // 2, body, 0)

    plsc.subcore_barrier()
    pltpu.sync_copy(acc.at[pl.ds(s * WB, WB)],
                    out_hbm.at[c, pl.ds(s * WB, WB), :])

    @pl.when(s == 0)
    def _():
        pltpu.sync_copy(acc.at[pl.ds(WB * NS, TAIL)],
                        out_hbm.at[c, pl.ds(WB * NS, TAIL), :])


def _tc_pre(x_ref, w_ref, degp_ref, hs_ref, dinv_ref):
    deg = 1.0 + degp_ref[0, :, 0:1] + degp_ref[1, :, 0:1]
    dinv = lax.rsqrt(deg)
    hp = jnp.dot(x_ref[...], w_ref[...], preferred_element_type=jnp.float32)
    hs_ref[...] = hp * dinv
    dinv_ref[...] = dinv


def _bn_input(accp_ref, hs_ref, dinv_ref, b_ref):
    y = (accp_ref[0] + accp_ref[1] + hs_ref[...]) * dinv_ref[...] + b_ref[...]
    mu = jnp.mean(y, axis=0, keepdims=True)
    var = jnp.mean((y - mu) ** 2, axis=0, keepdims=True)
    return y, mu, var


def _tc_mid(accp_ref, hs_ref, dinv_ref, b_ref, g_ref, be_ref, w_ref,
            hs2_ref):
    y, mu, var = _bn_input(accp_ref, hs_ref, dinv_ref, b_ref)
    h = (y - mu) * lax.rsqrt(var + 1e-5) * g_ref[...] + be_ref[...]
    h = jnp.maximum(h, 0.0)
    hp2 = jnp.dot(h, w_ref[...], preferred_element_type=jnp.float32)
    hs2_ref[...] = hp2 * dinv_ref[...]


def _tc_post(accp_ref, hs_ref, dinv_ref, b_ref, g_ref, be_ref, batch_ref,
             out_ref):
    y, mu, var = _bn_input(accp_ref, hs_ref, dinv_ref, b_ref)
    h = (y - mu) * lax.rsqrt(var + 1e-5) * g_ref[...] + be_ref[...]
    onehot = (lax.broadcasted_iota(jnp.int32, (G, N), 0)
              == batch_ref[...]).astype(jnp.float32)
    cnt = jnp.sum(onehot, axis=1, keepdims=True)
    pooled = jnp.dot(onehot, h, preferred_element_type=jnp.float32)
    out_ref[...] = pooled / jnp.maximum(cnt, 1.0)


_f32 = jnp.float32


def kernel(x, edge_index, batch, W1, b1, g1, be1, W2, b2, g2, be2,
           W3, b3, g3, be3):
    src4 = edge_index[0].reshape(NC, NS, CH, K).astype(jnp.int32)
    dst4 = edge_index[1].reshape(NC, NS, CH, K).astype(jnp.int32)
    dst3 = edge_index[1].reshape(NS, CHD, KD).astype(jnp.int32)
    batch2 = batch.reshape(1, N).astype(jnp.int32)
    b1r, b2r, b3r = (v.reshape(1, D) for v in (b1, b2, b3))
    g1r, g2r, g3r = (v.reshape(1, D) for v in (g1, g2, g3))
    be1r, be2r, be3r = (v.reshape(1, D) for v in (be1, be2, be3))

    degp = _sc_degree(dst3)

    hs1, dinv = pl.pallas_call(
        _tc_pre,
        out_shape=[jax.ShapeDtypeStruct((N, D), _f32),
                   jax.ShapeDtypeStruct((N, 1), _f32)],
    )(x, W1, degp)

    acc1 = _sc_propagate(hs1, src4, dst4)
    hs2 = pl.pallas_call(
        _tc_mid,
        out_shape=jax.ShapeDtypeStruct((N, D), _f32),
    )(acc1, hs1, dinv, b1r, g1r, be1r, W2)

    acc2 = _sc_propagate(hs2, src4, dst4)
    hs3 = pl.pallas_call(
        _tc_mid,
        out_shape=jax.ShapeDtypeStruct((N, D), _f32),
    )(acc2, hs2, dinv, b2r, g2r, be2r, W3)

    acc3 = _sc_propagate(hs3, src4, dst4)
    out = pl.pallas_call(
        _tc_post,
        out_shape=jax.ShapeDtypeStruct((G, D), _f32),
    )(acc3, hs3, dinv, b3r, g3r, be3r, batch2)
    return out
